# baseline (speedup 1.0000x reference)
import functools

import jax
import jax.numpy as jnp
from jax import lax
from jax.experimental import pallas as pl
from jax.experimental.pallas import tpu as pltpu
from jax.experimental.pallas import tpu_sc as plsc

F = 26
V = 100000
D = 32
B = 16384

NC, NS, L = 2, 16, 16
LA = 49920
LB = V - LA
G = 8192
NG = B // G
CH = G // 2
UN = 8


def _sc_col_gather(ttab, idx_t):
    mesh = plsc.VectorSubcoreMesh(core_axis_name="c", subcore_axis_name="s")

    @functools.partial(
        pl.kernel,
        mesh=mesh,
        out_type=jax.ShapeDtypeStruct((F * D, B), jnp.float32),
        compiler_params=pltpu.CompilerParams(needs_layout_passes=False),
        scratch_types=[
            pltpu.VMEM((LA,), jnp.float32),
            pltpu.VMEM((LB,), jnp.float32),
            pltpu.VMEM((G,), jnp.int32),
            pltpu.VMEM((G,), jnp.int32),
            pltpu.VMEM((G,), jnp.float32),
            pltpu.SemaphoreType.DMA,
            pltpu.SemaphoreType.DMA,
            pltpu.SemaphoreType.DMA,
            pltpu.SemaphoreType.DMA,
            pltpu.SemaphoreType.DMA,
        ],
    )
    def k(ttab_hbm, idx_hbm, out_hbm, bufa, bufb, ig0, ig1, outg,
          sa, sb, si0, si1, so):
        d = lax.axis_index("s") * NC + lax.axis_index("c")
        ig = (ig0, ig1)
        si = (si0, si1)

        def load_a(f):
            return pltpu.async_copy(ttab_hbm.at[f, d, pl.ds(0, LA)], bufa, sa)

        def load_b(f):
            return pltpu.async_copy(ttab_hbm.at[f, d, pl.ds(LA, LB)], bufb, sb)

        def load_idx(f, g):
            return pltpu.async_copy(
                idx_hbm.at[f, pl.ds(g * G, G)], ig[g % 2], si[g % 2])

        def flush(f, g, c):
            return pltpu.async_copy(
                outg.at[pl.ds(c * CH, CH)],
                out_hbm.at[f * D + d, pl.ds(g * G + c * CH, CH)], so)

        def pass_a(src):
            def step(i, _):
                for u in range(UN):
                    o = i * L * UN + u * L
                    iv = src[pl.ds(o, L)]
                    m = iv < LA
                    outg[pl.ds(o, L)] = plsc.load_gather(bufa, [iv], mask=m)
                return 0

            lax.fori_loop(0, G // (L * UN), step, 0)

        def pass_b(src, c):
            def step(i, _):
                for u in range(UN):
                    o = c * CH + i * L * UN + u * L
                    iv = src[pl.ds(o, L)]
                    m = iv >= LA
                    v = plsc.load_gather(bufb, [iv - LA], mask=m)
                    outg[pl.ds(o, L)] = jnp.where(m, v, outg[pl.ds(o, L)])
                return 0

            lax.fori_loop(0, CH // (L * UN), step, 0)

        load_idx(0, 0)
        cp_a0 = load_a(0)
        cp_b0 = load_b(0)
        cp_a0.wait()

        def per_field(f, _):
            for g in range(NG):
                pltpu.make_async_copy(
                    idx_hbm.at[f, pl.ds(g * G, G)], ig[g % 2],
                    si[g % 2]).wait()
                nf = jnp.minimum(f + (1 if g + 1 == NG else 0), F - 1)
                pltpu.async_copy(
                    idx_hbm.at[nf, pl.ds(((g + 1) % NG) * G, G)],
                    ig[(g + 1) % 2], si[(g + 1) % 2])

                @pl.when(jnp.logical_or(f > 0, g > 0))
                def _():
                    pltpu.make_async_copy(
                        outg.at[pl.ds(0, CH)],
                        out_hbm.at[0, pl.ds(0, CH)], so).wait()
                    pltpu.make_async_copy(
                        outg.at[pl.ds(0, CH)],
                        out_hbm.at[0, pl.ds(0, CH)], so).wait()

                pass_a(ig[g % 2])

                if g == 0:
                    pltpu.make_async_copy(
                        ttab_hbm.at[f, d, pl.ds(LA, LB)], bufb, sb).wait()

                if g + 1 == NG:
                    @pl.when(f + 1 < F)
                    def _():
                        load_a(f + 1)

                for c in range(2):
                    pass_b(ig[g % 2], c)
                    flush(f, g, c)

            @pl.when(f + 1 < F)
            def _():
                load_b(f + 1)
                pltpu.make_async_copy(
                    ttab_hbm.at[f, d, pl.ds(0, LA)], bufa, sa).wait()

            return 0

        lax.fori_loop(0, F, per_field, 0)

        pltpu.make_async_copy(
            idx_hbm.at[0, pl.ds(0, G)], ig[0], si[0]).wait()
        pltpu.make_async_copy(
            outg.at[pl.ds(0, CH)], out_hbm.at[0, pl.ds(0, CH)], so).wait()
        pltpu.make_async_copy(
            outg.at[pl.ds(0, CH)], out_hbm.at[0, pl.ds(0, CH)], so).wait()

    return k(ttab, idx_t)


def kernel(indices, tables):
    ttab = jnp.swapaxes(tables, 1, 2)
    idx_t = jnp.swapaxes(indices.astype(jnp.int32), 0, 1)
    out_t = _sc_col_gather(ttab, idx_t)
    return jnp.swapaxes(out_t, 0, 1)


# speedup vs baseline: 1.1001x; 1.1001x over previous
import functools

import jax
import jax.numpy as jnp
from jax import lax
from jax.experimental import pallas as pl
from jax.experimental.pallas import tpu as pltpu
from jax.experimental.pallas import tpu_sc as plsc

F = 26
V = 100000
D = 32
B = 16384

NC, NS, L = 2, 16, 16
NW = NC * NS
BCH = 4096
NBCH = B // BCH
UNROLL = 16


def _sc_col_gather(ttab, idx_t):
    mesh = plsc.VectorSubcoreMesh(core_axis_name="c", subcore_axis_name="s")

    @functools.partial(
        pl.kernel,
        mesh=mesh,
        out_type=jax.ShapeDtypeStruct((F * D, B), jnp.float32),
        compiler_params=pltpu.CompilerParams(needs_layout_passes=False),
        scratch_types=[
            pltpu.VMEM((V,), jnp.float32),
            pltpu.VMEM((BCH,), jnp.int32),
            pltpu.VMEM((BCH,), jnp.int32),
            pltpu.VMEM((BCH,), jnp.float32),
            pltpu.VMEM((BCH,), jnp.float32),
            pltpu.SemaphoreType.DMA,
            pltpu.SemaphoreType.DMA,
            pltpu.SemaphoreType.DMA,
            pltpu.SemaphoreType.DMA,
        ],
    )
    def k(ttab_hbm, idx_hbm, out_hbm, vrow,
          idxb0, idxb1, outb0, outb1, si0, si1, so0, so1):
        d = lax.axis_index("s") * NC + lax.axis_index("c")
        idxb = (idxb0, idxb1)
        outb = (outb0, outb1)
        si = (si0, si1)
        so = (so0, so1)

        def gather_chunk(src, dst):
            def gather_step(i, _):
                for u in range(UNROLL):
                    o = i * L * UNROLL + u * L
                    iv = src[pl.ds(o, L)]
                    dst[pl.ds(o, L)] = plsc.load_gather(vrow, [iv])
                return 0

            lax.fori_loop(0, BCH // (L * UNROLL), gather_step, 0)

        def per_field(f, _):
            row = f * D + d
            cp_i = [pltpu.async_copy(
                idx_hbm.at[f, pl.ds(0, BCH)], idxb[0], si[0]),
                pltpu.async_copy(
                idx_hbm.at[f, pl.ds(BCH, BCH)], idxb[1], si[1])]
            pltpu.sync_copy(ttab_hbm.at[f, d], vrow)
            cp_o = []
            for h in range(NBCH):
                cp_i[h].wait()
                if h >= 2:
                    cp_o[h - 2].wait()
                gather_chunk(idxb[h % 2], outb[h % 2])
                if h + 2 < NBCH:
                    cp_i.append(pltpu.async_copy(
                        idx_hbm.at[f, pl.ds((h + 2) * BCH, BCH)],
                        idxb[h % 2], si[h % 2]))
                cp_o.append(pltpu.async_copy(
                    outb[h % 2], out_hbm.at[row, pl.ds(h * BCH, BCH)],
                    so[h % 2]))
            cp_o[NBCH - 2].wait()
            cp_o[NBCH - 1].wait()
            return 0

        lax.fori_loop(0, F, per_field, 0)

    return k(ttab, idx_t)


def kernel(indices, tables):
    ttab = jnp.swapaxes(tables, 1, 2)
    idx_t = jnp.swapaxes(indices.astype(jnp.int32), 0, 1)
    out_t = _sc_col_gather(ttab, idx_t)
    return jnp.swapaxes(out_t, 0, 1)
